# baseline (device time: 11055 ns/iter reference)
import jax
import jax.numpy as jnp
from jax import lax
from jax.experimental import pallas as pl
from jax.experimental.pallas import tpu as pltpu


def kernel(x, dy, gamma):
    m, d = x.shape

    def body(x_ref, dy_ref, out_ref, partial_ref, comm_ref, send_sem, recv_sem):
        my_x = lax.axis_index("x")
        my_y = lax.axis_index("y")
        my_z = lax.axis_index("z")
        nbr = (my_x, 1 - my_y, my_z)

        barrier_sem = pltpu.get_barrier_semaphore()
        pl.semaphore_signal(
            barrier_sem, inc=1,
            device_id=nbr, device_id_type=pl.DeviceIdType.MESH,
        )

        xv = x_ref[:, :]
        dyv = dy_ref[:, :]
        ones_d = jnp.ones((d, 1), jnp.float32)

        s1 = jax.lax.dot(xv, ones_d)
        s2 = jax.lax.dot(xv * xv, ones_d)
        mu = s1 * (1.0 / d)
        rstd = lax.rsqrt(s2 * (1.0 / d) - mu * mu + 1e-5)

        w = jnp.concatenate(
            [-(rstd * mu), jnp.ones((m, 1), jnp.float32)], axis=1
        )
        td = jax.lax.dot(w.T, dyv)
        tg = jax.lax.dot(rstd.T, dyv * xv)
        partial_ref[0, :] = tg[0, :] + td[0, :]
        partial_ref[1, :] = td[1, :]

        pl.semaphore_wait(barrier_sem, 1)

        rdma = pltpu.make_async_remote_copy(
            src_ref=partial_ref,
            dst_ref=comm_ref,
            send_sem=send_sem,
            recv_sem=recv_sem,
            device_id=nbr,
            device_id_type=pl.DeviceIdType.MESH,
        )
        rdma.start()
        rdma.wait()

        out_ref[:, :] = partial_ref[:, :] + comm_ref[:, :]

    return pl.pallas_call(
        body,
        out_shape=jax.ShapeDtypeStruct((2, d), jnp.float32),
        in_specs=[
            pl.BlockSpec(memory_space=pltpu.VMEM),
            pl.BlockSpec(memory_space=pltpu.VMEM),
        ],
        out_specs=pl.BlockSpec(memory_space=pltpu.VMEM),
        scratch_shapes=[
            pltpu.VMEM((2, d), jnp.float32),
            pltpu.VMEM((2, d), jnp.float32),
            pltpu.SemaphoreType.DMA,
            pltpu.SemaphoreType.DMA,
        ],
        compiler_params=pltpu.CompilerParams(collective_id=0),
    )(x, dy)
